# baseline (speedup 1.0000x reference)
import jax
import jax.numpy as jnp
from jax.experimental import pallas as pl
from jax.experimental.pallas import tpu as pltpu

_MIN_BIN = 3.25
_MAX_BIN = 20.75
_NO_BINS = 15
_INF = 100000000.0
_CZ = 128
_N = 1024
_BI = 32
_CH = 8


def _embed_body(xi_ref, xjt_ref, sqb_ref, up_ref, w2_ref, b_ref, o_ref):
    xjt = xjt_ref[...]
    sqb = sqb_ref[...][0]
    up = up_ref[...][0]
    w2 = w2_ref[...]
    bias = b_ref[...]

    for h in range(_BI // _CH):
        xi = xi_ref[h * _CH:(h + 1) * _CH, :]
        d = None
        for c in range(3):
            df = xi[:, c:c + 1] - xjt[c:c + 1, :]
            sq = df * df
            d = sq if d is None else d + sq

        d3 = d[:, :, None]
        mask = (d3 > sqb) & (d3 < up)
        oh = mask.astype(jnp.float32).astype(jnp.bfloat16)
        oh2 = oh.reshape(_CH * _N, 2 * _NO_BINS)
        z = jnp.dot(oh2, w2, preferred_element_type=jnp.float32)
        o_ref[h * _CH * _N:(h + 1) * _CH * _N, :] = z + bias


def kernel(x, W, b):
    x2 = x[0]
    xjt = x2.T
    wt = W.T
    wh = wt.astype(jnp.bfloat16)
    wl = (wt - wh.astype(jnp.float32)).astype(jnp.bfloat16)
    w2 = jnp.concatenate([wh, wl], axis=0)
    b2 = b.reshape(1, _CZ)
    bins = jnp.linspace(_MIN_BIN, _MAX_BIN, _NO_BINS, dtype=x.dtype)
    sqb1 = (bins ** 2).reshape(1, _NO_BINS)
    up1 = jnp.concatenate(
        [sqb1[:, 1:], jnp.full((1, 1), _INF, x.dtype)], axis=1)
    sqb2 = jnp.concatenate([sqb1, sqb1], axis=1)
    up2 = jnp.concatenate([up1, up1], axis=1)

    out = pl.pallas_call(
        _embed_body,
        out_shape=jax.ShapeDtypeStruct((_N * _N, _CZ), jnp.float32),
        grid=(_N // _BI,),
        in_specs=[
            pl.BlockSpec((_BI, 3), lambda i: (i, 0)),
            pl.BlockSpec((3, _N), lambda i: (0, 0)),
            pl.BlockSpec((1, 2 * _NO_BINS), lambda i: (0, 0)),
            pl.BlockSpec((1, 2 * _NO_BINS), lambda i: (0, 0)),
            pl.BlockSpec((2 * _NO_BINS, _CZ), lambda i: (0, 0)),
            pl.BlockSpec((1, _CZ), lambda i: (0, 0)),
        ],
        out_specs=pl.BlockSpec((_BI * _N, _CZ), lambda i: (i, 0)),
        compiler_params=pltpu.CompilerParams(
            dimension_semantics=("arbitrary",),
            vmem_limit_bytes=64 * 1024 * 1024,
        ),
        name="ca_embed",
    )(x2, xjt, sqb2, up2, w2, b2)
    return out.reshape(1, _N, _N, _CZ)


# speedup vs baseline: 1.0195x; 1.0195x over previous
import jax
import jax.numpy as jnp
from jax.experimental import pallas as pl
from jax.experimental.pallas import tpu as pltpu

_MIN_BIN = 3.25
_MAX_BIN = 20.75
_NO_BINS = 15
_INF = 100000000.0
_CZ = 128
_N = 1024
_BI = 32
_CH = 32


def _embed_body(xi_ref, xjt_ref, sqb_ref, up_ref, w2_ref, b_ref, o_ref):
    xjt = xjt_ref[...]
    sqb = sqb_ref[...][0]
    up = up_ref[...][0]
    w2 = w2_ref[...]
    bias = b_ref[...]

    for h in range(_BI // _CH):
        xi = xi_ref[h * _CH:(h + 1) * _CH, :]
        d = None
        for c in range(3):
            df = xi[:, c:c + 1] - xjt[c:c + 1, :]
            sq = df * df
            d = sq if d is None else d + sq

        d3 = d[:, :, None]
        mask = (d3 > sqb) & (d3 < up)
        oh = mask.astype(jnp.float32).astype(jnp.bfloat16)
        oh2 = oh.reshape(_CH * _N, 2 * _NO_BINS)
        z = jnp.dot(oh2, w2, preferred_element_type=jnp.float32)
        o_ref[h * _CH * _N:(h + 1) * _CH * _N, :] = z + bias


def kernel(x, W, b):
    x2 = x[0]
    xjt = x2.T
    wt = W.T
    wh = wt.astype(jnp.bfloat16)
    wl = (wt - wh.astype(jnp.float32)).astype(jnp.bfloat16)
    w2 = jnp.concatenate([wh, wl], axis=0)
    b2 = b.reshape(1, _CZ)
    bins = jnp.linspace(_MIN_BIN, _MAX_BIN, _NO_BINS, dtype=x.dtype)
    sqb1 = (bins ** 2).reshape(1, _NO_BINS)
    up1 = jnp.concatenate(
        [sqb1[:, 1:], jnp.full((1, 1), _INF, x.dtype)], axis=1)
    sqb2 = jnp.concatenate([sqb1, sqb1], axis=1)
    up2 = jnp.concatenate([up1, up1], axis=1)

    out = pl.pallas_call(
        _embed_body,
        out_shape=jax.ShapeDtypeStruct((_N * _N, _CZ), jnp.float32),
        grid=(_N // _BI,),
        in_specs=[
            pl.BlockSpec((_BI, 3), lambda i: (i, 0)),
            pl.BlockSpec((3, _N), lambda i: (0, 0)),
            pl.BlockSpec((1, 2 * _NO_BINS), lambda i: (0, 0)),
            pl.BlockSpec((1, 2 * _NO_BINS), lambda i: (0, 0)),
            pl.BlockSpec((2 * _NO_BINS, _CZ), lambda i: (0, 0)),
            pl.BlockSpec((1, _CZ), lambda i: (0, 0)),
        ],
        out_specs=pl.BlockSpec((_BI * _N, _CZ), lambda i: (i, 0)),
        compiler_params=pltpu.CompilerParams(
            dimension_semantics=("arbitrary",),
            vmem_limit_bytes=64 * 1024 * 1024,
        ),
        name="ca_embed",
    )(x2, xjt, sqb2, up2, w2, b2)
    return out.reshape(1, _N, _N, _CZ)
